# baseline (speedup 1.0000x reference)
import jax
import jax.numpy as jnp
from jax.experimental import pallas as pl
from jax.experimental.pallas import tpu as pltpu


def _gcn_kernel(x_ref, w_ref, b_ref, adj_a_ref, adj_b_ref, out_ref, support_ref):
    @pl.when(pl.program_id(0) == 0)
    def _():
        support_ref[...] = jnp.dot(
            x_ref[...], w_ref[...], preferred_element_type=jnp.float32
        )

    out_ref[0] = (
        jnp.dot(adj_a_ref[0], support_ref[...], preferred_element_type=jnp.float32)
        + b_ref[...]
    )
    out_ref[1] = (
        jnp.dot(adj_b_ref[0], support_ref[...], preferred_element_type=jnp.float32)
        + b_ref[...]
    )


def kernel(x, adj, W, b):
    n, din = x.shape
    dout = W.shape[1]
    bm = 200
    nh = n // 2
    b2 = b.reshape(1, dout)
    adj3 = adj.reshape(2, nh, n)
    out = pl.pallas_call(
        _gcn_kernel,
        grid=(nh // bm,),
        in_specs=[
            pl.BlockSpec((n, din), lambda m: (0, 0)),
            pl.BlockSpec((din, dout), lambda m: (0, 0)),
            pl.BlockSpec((1, dout), lambda m: (0, 0)),
            pl.BlockSpec((1, bm, n), lambda m: (0, m, 0)),
            pl.BlockSpec((1, bm, n), lambda m: (1, m, 0)),
        ],
        out_specs=pl.BlockSpec((2, bm, dout), lambda m: (0, m, 0)),
        out_shape=jax.ShapeDtypeStruct((2, nh, dout), jnp.float32),
        scratch_shapes=[pltpu.VMEM((n, dout), jnp.float32)],
    )(x, W, b2, adj3, adj3)
    return out.reshape(n, dout)


# speedup vs baseline: 1.0129x; 1.0129x over previous
import jax
import jax.numpy as jnp
from jax.experimental import pallas as pl
from jax.experimental.pallas import tpu as pltpu


def _gcn_kernel(x_ref, w_ref, b_ref, adj_ref, out_ref, support_ref):
    @pl.when(pl.program_id(0) == 0)
    def _():
        support_ref[...] = jnp.dot(
            x_ref[...], w_ref[...], preferred_element_type=jnp.float32
        )

    out_ref[...] = (
        jnp.dot(adj_ref[...], support_ref[...], preferred_element_type=jnp.float32)
        + b_ref[...]
    )


def kernel(x, adj, W, b):
    n, din = x.shape
    dout = W.shape[1]
    bm = 200
    b2 = b.reshape(1, dout)
    return pl.pallas_call(
        _gcn_kernel,
        grid=(n // bm,),
        in_specs=[
            pl.BlockSpec((n, din), lambda m: (0, 0)),
            pl.BlockSpec((din, dout), lambda m: (0, 0)),
            pl.BlockSpec((1, dout), lambda m: (0, 0)),
            pl.BlockSpec((bm, n), lambda m: (m, 0)),
        ],
        out_specs=pl.BlockSpec((bm, dout), lambda m: (m, 0)),
        out_shape=jax.ShapeDtypeStruct((n, dout), jnp.float32),
        scratch_shapes=[pltpu.VMEM((n, dout), jnp.float32)],
    )(x, W, b2, adj)


# speedup vs baseline: 1.0171x; 1.0041x over previous
import jax
import jax.numpy as jnp
from jax.experimental import pallas as pl
from jax.experimental.pallas import tpu as pltpu


def _gcn_kernel(x_ref, w_ref, b_ref, adj_ref, out_ref, support_ref):
    @pl.when(pl.program_id(0) == 0)
    def _():
        support_ref[...] = jnp.dot(
            x_ref[...], w_ref[...], preferred_element_type=jnp.float32
        )

    out_ref[...] = (
        jnp.dot(adj_ref[...], support_ref[...], preferred_element_type=jnp.float32)
        + b_ref[...]
    )


def kernel(x, adj, W, b):
    n, din = x.shape
    dout = W.shape[1]
    bm = 400
    b2 = b.reshape(1, dout)
    return pl.pallas_call(
        _gcn_kernel,
        grid=(n // bm,),
        in_specs=[
            pl.BlockSpec((n, din), lambda m: (0, 0)),
            pl.BlockSpec((din, dout), lambda m: (0, 0)),
            pl.BlockSpec((1, dout), lambda m: (0, 0)),
            pl.BlockSpec((bm, n), lambda m: (m, 0)),
        ],
        out_specs=pl.BlockSpec((bm, dout), lambda m: (m, 0)),
        out_shape=jax.ShapeDtypeStruct((n, dout), jnp.float32),
        scratch_shapes=[pltpu.VMEM((n, dout), jnp.float32)],
    )(x, W, b2, adj)
